# Initial kernel scaffold; baseline (speedup 1.0000x reference)
import jax
import jax.numpy as jnp
from jax.experimental import pallas as pl


def kernel(z, W):
    raise NotImplementedError("write your pallas kernel here")


# speedup vs baseline: 1.2333x; 1.2333x over previous
import functools

import jax
import jax.numpy as jnp
from jax import lax
from jax.experimental import pallas as pl
from jax.experimental.pallas import tpu as pltpu
from jax.experimental.pallas import tpu_sc as plsc

_NE = 2048
_D = 128
_BETA = 0.25
_R_BLK = 512


def _tc_dist_argmin(flat, W):
    n_rows = flat.shape[0]
    n_blk = n_rows // _R_BLK

    def body(z_ref, w_ref, idx_ref, loss_ref):
        zb = z_ref[...]
        w = w_ref[...]
        zsq = jnp.sum(zb ** 2, axis=1, keepdims=True)
        wsq = jnp.sum(w ** 2, axis=1)
        mm = lax.dot_general(zb, w, (((1,), (1,)), ((), ())),
                             preferred_element_type=jnp.float32)
        dist = zsq - 2.0 * mm + wsq[None, :]
        minv = jnp.min(dist, axis=1, keepdims=True)
        ids = lax.broadcasted_iota(jnp.int32, dist.shape, 1)
        idx = jnp.min(jnp.where(dist == minv, ids, _NE), axis=1)
        idx_ref[0, 0, :] = idx

        @pl.when(pl.program_id(0) == 0)
        def _init():
            loss_ref[0, 0] = 0.0

        loss_ref[0, 0] += jnp.sum(minv)

        @pl.when(pl.program_id(0) == pl.num_programs(0) - 1)
        def _finalize():
            a = loss_ref[0, 0] / (n_rows * _D)
            loss_ref[0, 0] = a + _BETA * a

    idx3, loss = pl.pallas_call(
        body,
        grid=(n_blk,),
        in_specs=[pl.BlockSpec((_R_BLK, _D), lambda i: (i, 0)),
                  pl.BlockSpec((_NE, _D), lambda i: (0, 0))],
        out_specs=[pl.BlockSpec((1, 1, _R_BLK), lambda i: (i, 0, 0)),
                   pl.BlockSpec(memory_space=pltpu.SMEM)],
        out_shape=[jax.ShapeDtypeStruct((n_blk, 1, _R_BLK), jnp.int32),
                   jax.ShapeDtypeStruct((1, 1), jnp.float32)],
    )(flat, W)
    return idx3.reshape(n_rows), loss[0, 0]


def _sc_gather(W, idx):
    info = plsc.get_sparse_core_info()
    nc, ns = info.num_cores, info.num_subcores
    nw = nc * ns
    n = idx.shape[0]
    bpw = n // nw
    nch = 3
    ch = bpw // nch
    idx3 = idx.reshape(nw, nch, ch)
    mesh = plsc.VectorSubcoreMesh(core_axis_name="c", subcore_axis_name="s")

    @functools.partial(
        pl.kernel, mesh=mesh,
        out_type=jax.ShapeDtypeStruct((n, _D), jnp.float32),
        scratch_types=[pltpu.VMEM((nch, ch), jnp.int32),
                       pltpu.VMEM((bpw, _D), jnp.float32),
                       pltpu.SemaphoreType.DMA],
    )
    def gk(table_hbm, idx_hbm, out_hbm, idx_v, rows_v, sem):
        wid = lax.axis_index("s") * nc + lax.axis_index("c")
        pltpu.sync_copy(idx_hbm.at[wid], idx_v)
        copies = [pltpu.async_copy(table_hbm.at[idx_v.at[j]],
                                   rows_v.at[pl.ds(j * ch, ch)], sem)
                  for j in range(nch)]
        for c in copies:
            c.wait()
        pltpu.sync_copy(rows_v, out_hbm.at[pl.ds(wid * bpw, bpw)])

    return gk(W, idx3)


def kernel(z, W):
    b, s, d = z.shape
    n_rows = b * s
    flat = z.reshape(n_rows, d)
    idx, loss = _tc_dist_argmin(flat, W)
    q = _sc_gather(W, idx)
    return q.reshape(z.shape), idx.reshape(b, s), loss


# speedup vs baseline: 1.3817x; 1.1204x over previous
import functools

import jax
import jax.numpy as jnp
from jax import lax
from jax.experimental import pallas as pl
from jax.experimental.pallas import tpu as pltpu
from jax.experimental.pallas import tpu_sc as plsc

_NE = 2048
_D = 128
_BETA = 0.25
_R_BLK = 512


def _tc_dist_argmin(flat, W):
    n_rows = flat.shape[0]
    n_blk = n_rows // _R_BLK
    n_ch = _NE // _D

    def body(z_ref, w_ref, idx_ref, loss_ref, n2w_ref, wsq_ref):
        @pl.when(pl.program_id(0) == 0)
        def _prep():
            w = w_ref[...]
            n2w_ref[...] = w * -2.0
            wsq_ref[...] = jnp.sum(w ** 2, axis=1).reshape(1, _NE)
            loss_ref[0, 0] = 0.0

        zb = z_ref[...]
        zsq = jnp.sum(zb ** 2, axis=1, keepdims=True)
        mmn = lax.dot_general(zb, n2w_ref[...], (((1,), (1,)), ((), ())),
                              preferred_element_type=jnp.float32)
        lane = lax.broadcasted_iota(jnp.int32, (_R_BLK, _D), 1)
        v = (zsq + mmn[:, :_D]) + wsq_ref[:, :_D]
        c = lane
        for ch in range(1, n_ch):
            sl = slice(ch * _D, (ch + 1) * _D)
            vc = (zsq + mmn[:, sl]) + wsq_ref[:, sl]
            take = vc < v
            v = jnp.where(take, vc, v)
            c = jnp.where(take, lane + ch * _D, c)
        minv = jnp.min(v, axis=1, keepdims=True)
        idx = jnp.min(jnp.where(v == minv, c, _NE), axis=1)
        idx_ref[0, 0, :] = idx

        loss_ref[0, 0] += jnp.sum(minv)

        @pl.when(pl.program_id(0) == pl.num_programs(0) - 1)
        def _finalize():
            a = loss_ref[0, 0] / (n_rows * _D)
            loss_ref[0, 0] = a + _BETA * a

    idx3, loss = pl.pallas_call(
        body,
        grid=(n_blk,),
        in_specs=[pl.BlockSpec((_R_BLK, _D), lambda i: (i, 0)),
                  pl.BlockSpec((_NE, _D), lambda i: (0, 0))],
        out_specs=[pl.BlockSpec((1, 1, _R_BLK), lambda i: (i, 0, 0)),
                   pl.BlockSpec(memory_space=pltpu.SMEM)],
        out_shape=[jax.ShapeDtypeStruct((n_blk, 1, _R_BLK), jnp.int32),
                   jax.ShapeDtypeStruct((1, 1), jnp.float32)],
        scratch_shapes=[pltpu.VMEM((_NE, _D), jnp.float32),
                        pltpu.VMEM((1, _NE), jnp.float32)],
    )(flat, W)
    return idx3.reshape(n_rows), loss[0, 0]


def _sc_gather(W, idx):
    info = plsc.get_sparse_core_info()
    nc, ns = info.num_cores, info.num_subcores
    nw = nc * ns
    n = idx.shape[0]
    bpw = n // nw
    nch = 3
    ch = bpw // nch
    idx3 = idx.reshape(nw, nch, ch)
    mesh = plsc.VectorSubcoreMesh(core_axis_name="c", subcore_axis_name="s")

    @functools.partial(
        pl.kernel, mesh=mesh,
        out_type=jax.ShapeDtypeStruct((n, _D), jnp.float32),
        scratch_types=[pltpu.VMEM((nch, ch), jnp.int32),
                       pltpu.VMEM((bpw, _D), jnp.float32),
                       pltpu.SemaphoreType.DMA],
    )
    def gk(table_hbm, idx_hbm, out_hbm, idx_v, rows_v, sem):
        wid = lax.axis_index("s") * nc + lax.axis_index("c")
        pltpu.sync_copy(idx_hbm.at[wid], idx_v)
        copies = [pltpu.async_copy(table_hbm.at[idx_v.at[j]],
                                   rows_v.at[pl.ds(j * ch, ch)], sem)
                  for j in range(nch)]
        for c in copies:
            c.wait()
        pltpu.sync_copy(rows_v, out_hbm.at[pl.ds(wid * bpw, bpw)])

    return gk(W, idx3)


def kernel(z, W):
    b, s, d = z.shape
    n_rows = b * s
    flat = z.reshape(n_rows, d)
    idx, loss = _tc_dist_argmin(flat, W)
    q = _sc_gather(W, idx)
    return q.reshape(z.shape), idx.reshape(b, s), loss


# speedup vs baseline: 1.3833x; 1.0012x over previous
import functools

import jax
import jax.numpy as jnp
from jax import lax
from jax.experimental import pallas as pl
from jax.experimental.pallas import tpu as pltpu
from jax.experimental.pallas import tpu_sc as plsc

_NE = 2048
_D = 128
_BETA = 0.25
_R_BLK = 1024


def _tc_dist_argmin(flat, W):
    n_rows = flat.shape[0]
    n_blk = n_rows // _R_BLK
    n_ch = _NE // _D

    def body(z_ref, w_ref, idx_ref, loss_ref, n2w_ref, wsq_ref):
        @pl.when(pl.program_id(0) == 0)
        def _prep():
            w = w_ref[...]
            n2w_ref[...] = w * -2.0
            wsq_ref[...] = jnp.sum(w ** 2, axis=1).reshape(1, _NE)
            loss_ref[0, 0] = 0.0

        zb = z_ref[...]
        zsq = jnp.sum(zb ** 2, axis=1, keepdims=True)
        mmn = lax.dot_general(zb, n2w_ref[...], (((1,), (1,)), ((), ())),
                              preferred_element_type=jnp.float32)
        lane = lax.broadcasted_iota(jnp.int32, (_R_BLK, _D), 1)
        v = (zsq + mmn[:, :_D]) + wsq_ref[:, :_D]
        c = lane
        for ch in range(1, n_ch):
            sl = slice(ch * _D, (ch + 1) * _D)
            vc = (zsq + mmn[:, sl]) + wsq_ref[:, sl]
            take = vc < v
            v = jnp.where(take, vc, v)
            c = jnp.where(take, lane + ch * _D, c)
        minv = jnp.min(v, axis=1, keepdims=True)
        idx = jnp.min(jnp.where(v == minv, c, _NE), axis=1)
        idx_ref[0, 0, :] = idx

        loss_ref[0, 0] += jnp.sum(minv)

        @pl.when(pl.program_id(0) == pl.num_programs(0) - 1)
        def _finalize():
            a = loss_ref[0, 0] / (n_rows * _D)
            loss_ref[0, 0] = a + _BETA * a

    idx3, loss = pl.pallas_call(
        body,
        grid=(n_blk,),
        in_specs=[pl.BlockSpec((_R_BLK, _D), lambda i: (i, 0)),
                  pl.BlockSpec((_NE, _D), lambda i: (0, 0))],
        out_specs=[pl.BlockSpec((1, 1, _R_BLK), lambda i: (i, 0, 0)),
                   pl.BlockSpec(memory_space=pltpu.SMEM)],
        out_shape=[jax.ShapeDtypeStruct((n_blk, 1, _R_BLK), jnp.int32),
                   jax.ShapeDtypeStruct((1, 1), jnp.float32)],
        scratch_shapes=[pltpu.VMEM((_NE, _D), jnp.float32),
                        pltpu.VMEM((1, _NE), jnp.float32)],
    )(flat, W)
    return idx3.reshape(n_rows), loss[0, 0]


def _sc_gather(W, idx):
    info = plsc.get_sparse_core_info()
    nc, ns = info.num_cores, info.num_subcores
    nw = nc * ns
    n = idx.shape[0]
    bpw = n // nw
    nch = 3
    ch = bpw // nch
    idx3 = idx.reshape(nw, nch, ch)
    mesh = plsc.VectorSubcoreMesh(core_axis_name="c", subcore_axis_name="s")

    @functools.partial(
        pl.kernel, mesh=mesh,
        out_type=jax.ShapeDtypeStruct((n, _D), jnp.float32),
        scratch_types=[pltpu.VMEM((nch, ch), jnp.int32),
                       pltpu.VMEM((bpw, _D), jnp.float32),
                       pltpu.SemaphoreType.DMA],
    )
    def gk(table_hbm, idx_hbm, out_hbm, idx_v, rows_v, sem):
        wid = lax.axis_index("s") * nc + lax.axis_index("c")
        pltpu.sync_copy(idx_hbm.at[wid], idx_v)
        copies = [pltpu.async_copy(table_hbm.at[idx_v.at[j]],
                                   rows_v.at[pl.ds(j * ch, ch)], sem)
                  for j in range(nch)]
        for c in copies:
            c.wait()
        pltpu.sync_copy(rows_v, out_hbm.at[pl.ds(wid * bpw, bpw)])

    return gk(W, idx3)


def kernel(z, W):
    b, s, d = z.shape
    n_rows = b * s
    flat = z.reshape(n_rows, d)
    idx, loss = _tc_dist_argmin(flat, W)
    q = _sc_gather(W, idx)
    return q.reshape(z.shape), idx.reshape(b, s), loss
